# baseline (speedup 1.0000x reference)
import jax
import jax.numpy as jnp
from jax.experimental import pallas as pl
from jax.experimental.pallas import tpu as pltpu

TH = 64
HB = 8


def _gmul(a, m):
    R, M = a.shape
    return (a.reshape(R // 8, 8, M) * m[None]).reshape(R, M)


def _body(xm_ref, xu_ref, xd_ref, mn_ref, mu_ref, md_ref, mm_ref,
          w1_ref, b1_ref, w2_ref, b2_ref, w3_ref, b3_ref, o_ref):
    t = pl.program_id(1)
    nt = pl.num_programs(1)
    C = xm_ref.shape[1]
    Th, W = xm_ref.shape[2], xm_ref.shape[3]
    N = Th * W
    Cm = w1_ref.shape[0]
    Ne = N + 2 * W

    xm = xm_ref[0]
    mn = mn_ref[0]
    m8 = mm_ref[0]

    xq = (xm * mn[None]).astype(jnp.bfloat16).reshape(C, N)

    t1m = jnp.dot(w1_ref[...], xq, preferred_element_type=jnp.float32)
    t1m = _gmul(jnp.maximum(t1m + b1_ref[...], 0.0), m8)

    xu = xu_ref[0, 0]
    xd = xd_ref[0, 0]
    mu8 = mu_ref[0, 0]
    md8 = md_ref[0, 0]
    t1u = jnp.dot(w1_ref[...], _gmul(xu, mu8).astype(jnp.bfloat16),
                  preferred_element_type=jnp.float32)
    t1u = _gmul(jnp.maximum(t1u + b1_ref[...], 0.0), mu8)
    t1u = t1u * jnp.where(t == 0, 0.0, 1.0)
    t1d = jnp.dot(w1_ref[...], _gmul(xd, md8).astype(jnp.bfloat16),
                  preferred_element_type=jnp.float32)
    t1d = _gmul(jnp.maximum(t1d + b1_ref[...], 0.0), md8)
    t1d = t1d * jnp.where(t == nt - 1, 0.0, 1.0)

    t1e = jnp.concatenate([t1u, t1m, t1d], axis=1).astype(jnp.bfloat16)

    col = jax.lax.broadcasted_iota(jnp.int32, (8, Ne), 1) % W
    zR = jnp.where(col == 0, 0.0, 1.0).astype(jnp.bfloat16)
    zL = jnp.where(col == W - 1, 0.0, 1.0).astype(jnp.bfloat16)
    zero1 = jnp.zeros((Cm, 1), jnp.bfloat16)
    t1eR = _gmul(jnp.concatenate([zero1, t1e[:, :Ne - 1]], axis=1), zR)
    t1eL = _gmul(jnp.concatenate([t1e[:, 1:], zero1], axis=1), zL)

    tall = jnp.concatenate([t1eR, t1e, t1eL], axis=0)
    acc = b2_ref[...] * jnp.ones((Cm, N), jnp.float32)
    for dy in range(3):
        acc = acc + jnp.dot(w2_ref[dy], tall[:, dy * W:dy * W + N],
                            preferred_element_type=jnp.float32)
    t2 = _gmul(jnp.maximum(acc, 0.0), m8).astype(jnp.bfloat16)

    y3 = jnp.dot(w3_ref[...], t2, preferred_element_type=jnp.float32) + b3_ref[...]
    y3n = y3.astype(jnp.bfloat16).reshape(C, Th, W).astype(jnp.float32)
    o_ref[0] = jnp.maximum(y3n + xm, 0.0)


def kernel(x, mask, w1, g1, b1, rm1, rv1, w2, g2, b2, rm2, rv2,
           w3, g3, b3, rm3, rv3, inference=False):
    B, C, H, W = x.shape
    Cm = w1.shape[0]
    mh, mw = mask.shape[2], mask.shape[3]
    N = TH * W

    s1 = g1 / jnp.sqrt(rv1 + 1e-5)
    s2 = g2 / jnp.sqrt(rv2 + 1e-5)
    s3 = g3 / jnp.sqrt(rv3 + 1e-5)
    w1f = (w1[:, :, 0, 0] * s1[:, None]).astype(jnp.bfloat16)
    b1f = (b1 - rm1 * s1)[:, None]
    w2s = w2 * s2[:, None, None, None]
    w2f = jnp.concatenate([w2s[:, :, :, 0], w2s[:, :, :, 1], w2s[:, :, :, 2]],
                          axis=1)
    w2f = jnp.transpose(w2f, (2, 0, 1)).astype(jnp.bfloat16)
    b2f = (b2 - rm2 * s2)[:, None]
    w3f = (w3[:, :, 0, 0] * s3[:, None]).astype(jnp.bfloat16)
    b3f = (b3 - rm3 * s3)[:, None]

    mnat = jnp.broadcast_to(mask[:, 0, :, None, :, None],
                            (B, mh, H // mh, mw, W // mw)).reshape(B, H, W)
    m8 = jnp.broadcast_to(mnat.reshape(B, 1, H * W), (B, 8, H * W))

    nt = H // TH
    up_rows = [max(t * TH - 1, 0) for t in range(nt)]
    dn_rows = [min(t * TH + TH, H - 1) for t in range(nt)]
    ridx = jnp.array(up_rows + dn_rows, dtype=jnp.int32)
    xhalo = jnp.transpose(x[:, :, ridx, :], (0, 2, 1, 3))
    mhalo = jnp.broadcast_to(mnat[:, ridx, None, :], (B, 2 * nt, 8, W))

    grid = (B, nt)

    out = pl.pallas_call(
        _body,
        grid=grid,
        in_specs=[
            pl.BlockSpec((1, C, TH, W), lambda b, t: (b, 0, t, 0)),
            pl.BlockSpec((1, 1, C, W), lambda b, t: (b, t, 0, 0)),
            pl.BlockSpec((1, 1, C, W), lambda b, t: (b, nt + t, 0, 0)),
            pl.BlockSpec((1, TH, W), lambda b, t: (b, t, 0)),
            pl.BlockSpec((1, 1, 8, W), lambda b, t: (b, t, 0, 0)),
            pl.BlockSpec((1, 1, 8, W), lambda b, t: (b, nt + t, 0, 0)),
            pl.BlockSpec((1, 8, N), lambda b, t: (b, 0, t)),
            pl.BlockSpec((Cm, C), lambda b, t: (0, 0)),
            pl.BlockSpec((Cm, 1), lambda b, t: (0, 0)),
            pl.BlockSpec((3, Cm, 3 * Cm), lambda b, t: (0, 0, 0)),
            pl.BlockSpec((Cm, 1), lambda b, t: (0, 0)),
            pl.BlockSpec((C, Cm), lambda b, t: (0, 0)),
            pl.BlockSpec((C, 1), lambda b, t: (0, 0)),
        ],
        out_specs=pl.BlockSpec((1, C, TH, W), lambda b, t: (b, 0, t, 0)),
        out_shape=jax.ShapeDtypeStruct((B, C, H, W), jnp.float32),
        compiler_params=pltpu.CompilerParams(
            dimension_semantics=("parallel", "arbitrary")),
    )(x, xhalo, xhalo, mnat, mhalo, mhalo, m8, w1f, b1f, w2f, b2f, w3f, b3f)
    return out


# speedup vs baseline: 1.0338x; 1.0338x over previous
import jax
import jax.numpy as jnp
from jax.experimental import pallas as pl
from jax.experimental.pallas import tpu as pltpu

TH = 64
HB = 8


def _gmul(a, m):
    R, M = a.shape
    return (a.reshape(R // 8, 8, M) * m[None]).reshape(R, M)


def _body(xm_ref, xu_ref, xd_ref, mn_ref, mu_ref, md_ref, mm_ref,
          w1_ref, b1_ref, w2_ref, b2_ref, w3_ref, b3_ref, o_ref):
    t = pl.program_id(1)
    nt = pl.num_programs(1)
    C = xm_ref.shape[1]
    Th, W = xm_ref.shape[2], xm_ref.shape[3]
    N = Th * W
    Cm = w1_ref.shape[0]
    Ne = N + 2 * W

    xm = xm_ref[0]
    mn = mn_ref[0]
    m8 = mm_ref[0]

    xq = (xm * mn[None]).astype(jnp.bfloat16).reshape(C, N)

    t1m = jnp.dot(w1_ref[...], xq, preferred_element_type=jnp.float32)
    t1m = _gmul(jnp.maximum(t1m + b1_ref[...], 0.0), m8)

    xu = xu_ref[0, :, HB - 1]
    xd = xd_ref[0, :, 0]
    mu = mu_ref[0, HB - 1].reshape(1, W)
    md = md_ref[0, 0].reshape(1, W)
    t1u = jnp.dot(w1_ref[...], (xu * mu).astype(jnp.bfloat16),
                  preferred_element_type=jnp.float32)
    t1u = jnp.maximum(t1u + b1_ref[...], 0.0) * mu
    t1u = t1u * jnp.where(t == 0, 0.0, 1.0)
    t1d = jnp.dot(w1_ref[...], (xd * md).astype(jnp.bfloat16),
                  preferred_element_type=jnp.float32)
    t1d = jnp.maximum(t1d + b1_ref[...], 0.0) * md
    t1d = t1d * jnp.where(t == nt - 1, 0.0, 1.0)

    t1e = jnp.concatenate([t1u, t1m, t1d], axis=1).astype(jnp.bfloat16)

    col = jax.lax.broadcasted_iota(jnp.int32, (8, Ne), 1) % W
    zR = jnp.where(col == 0, 0.0, 1.0).astype(jnp.bfloat16)
    zL = jnp.where(col == W - 1, 0.0, 1.0).astype(jnp.bfloat16)
    zero1 = jnp.zeros((Cm, 1), jnp.bfloat16)
    t1eR = _gmul(jnp.concatenate([zero1, t1e[:, :Ne - 1]], axis=1), zR)
    t1eL = _gmul(jnp.concatenate([t1e[:, 1:], zero1], axis=1), zL)

    tall = jnp.concatenate([t1eR, t1e, t1eL], axis=0)
    acc = b2_ref[...] * jnp.ones((Cm, N), jnp.float32)
    for dy in range(3):
        acc = acc + jnp.dot(w2_ref[dy], tall[:, dy * W:dy * W + N],
                            preferred_element_type=jnp.float32)
    t2 = _gmul(jnp.maximum(acc, 0.0), m8).astype(jnp.bfloat16)

    y3 = jnp.dot(w3_ref[...], t2, preferred_element_type=jnp.float32) + b3_ref[...]
    y3n = y3.astype(jnp.bfloat16).reshape(C, Th, W).astype(jnp.float32)
    o_ref[0] = jnp.maximum(y3n + xm, 0.0)


def kernel(x, mask, w1, g1, b1, rm1, rv1, w2, g2, b2, rm2, rv2,
           w3, g3, b3, rm3, rv3, inference=False):
    B, C, H, W = x.shape
    Cm = w1.shape[0]
    mh, mw = mask.shape[2], mask.shape[3]
    N = TH * W

    s1 = g1 / jnp.sqrt(rv1 + 1e-5)
    s2 = g2 / jnp.sqrt(rv2 + 1e-5)
    s3 = g3 / jnp.sqrt(rv3 + 1e-5)
    w1f = (w1[:, :, 0, 0] * s1[:, None]).astype(jnp.bfloat16)
    b1f = (b1 - rm1 * s1)[:, None]
    w2s = w2 * s2[:, None, None, None]
    w2f = jnp.concatenate([w2s[:, :, :, 0], w2s[:, :, :, 1], w2s[:, :, :, 2]],
                          axis=1)
    w2f = jnp.transpose(w2f, (2, 0, 1)).astype(jnp.bfloat16)
    b2f = (b2 - rm2 * s2)[:, None]
    w3f = (w3[:, :, 0, 0] * s3[:, None]).astype(jnp.bfloat16)
    b3f = (b3 - rm3 * s3)[:, None]

    mnat = jnp.broadcast_to(mask[:, 0, :, None, :, None],
                            (B, mh, H // mh, mw, W // mw)).reshape(B, H, W)
    m8 = jnp.broadcast_to(mnat.reshape(B, 1, H * W), (B, 8, H * W))

    nt = H // TH
    nhb = H // HB
    rb = TH // HB
    grid = (B, nt)

    out = pl.pallas_call(
        _body,
        grid=grid,
        in_specs=[
            pl.BlockSpec((1, C, TH, W), lambda b, t: (b, 0, t, 0)),
            pl.BlockSpec((1, C, HB, W), lambda b, t: (b, 0, jnp.maximum(t * rb - 1, 0), 0)),
            pl.BlockSpec((1, C, HB, W), lambda b, t: (b, 0, jnp.minimum(t * rb + rb, nhb - 1), 0)),
            pl.BlockSpec((1, TH, W), lambda b, t: (b, t, 0)),
            pl.BlockSpec((1, HB, W), lambda b, t: (b, jnp.maximum(t * rb - 1, 0), 0)),
            pl.BlockSpec((1, HB, W), lambda b, t: (b, jnp.minimum(t * rb + rb, nhb - 1), 0)),
            pl.BlockSpec((1, 8, N), lambda b, t: (b, 0, t)),
            pl.BlockSpec((Cm, C), lambda b, t: (0, 0)),
            pl.BlockSpec((Cm, 1), lambda b, t: (0, 0)),
            pl.BlockSpec((3, Cm, 3 * Cm), lambda b, t: (0, 0, 0)),
            pl.BlockSpec((Cm, 1), lambda b, t: (0, 0)),
            pl.BlockSpec((C, Cm), lambda b, t: (0, 0)),
            pl.BlockSpec((C, 1), lambda b, t: (0, 0)),
        ],
        out_specs=pl.BlockSpec((1, C, TH, W), lambda b, t: (b, 0, t, 0)),
        out_shape=jax.ShapeDtypeStruct((B, C, H, W), jnp.float32),
        compiler_params=pltpu.CompilerParams(
            dimension_semantics=("parallel", "arbitrary")),
    )(x, x, x, mnat, mnat, mnat, m8, w1f, b1f, w2f, b2f, w3f, b3f)
    return out
